# baseline (speedup 1.0000x reference)
import functools

import jax
import jax.numpy as jnp
from jax import lax
from jax.experimental import pallas as pl
from jax.experimental.pallas import tpu as pltpu
from jax.experimental.pallas import tpu_sc as plsc

NC, NS, L = 2, 16, 16
NEG = -3.0e38
BIG = 10**9


def _scores_body(q_ref, w1_ref, b1_ref, vw_ref, vb_ref, out_ref, *, rows,
                 t_pad):
    for i in range(rows):
        h = jnp.tanh(jnp.dot(q_ref[i], w1_ref[...],
                             preferred_element_type=jnp.float32) + b1_ref[0])
        s = lax.dot_general(vw_ref[...], h, (((1,), (1,)), ((), ())),
                            preferred_element_type=jnp.float32)
        out_ref[i, :] = jnp.concatenate(
            [s[0] + vb_ref[0, 0], jnp.full((t_pad,), NEG, jnp.float32)])


def _nms_one_row(r, raw_hbm, cv_hbm, ci_hbm, row_v, alive_v, chosen_v, cv_v,
                 ci_v, *, T, C, radius, nstep, kslot):
    nch = row_v.shape[0] // L
    pltpu.sync_copy(raw_hbm.at[r], row_v)
    iota = lax.iota(jnp.int32, L)
    ones = jnp.ones((L,), jnp.float32)
    zeros = jnp.zeros((L,), jnp.float32)
    for j in range(nch):
        sl = pl.ds(j * L, L)
        alive_v[sl] = ones
        chosen_v[sl] = zeros

    def step(_, carry):
        mvec = jnp.full((L,), NEG, jnp.float32)
        for j in range(nch):
            sl = pl.ds(j * L, L)
            cand = jnp.where((alive_v[sl] > 0) & (chosen_v[sl] == 0),
                             row_v[sl], NEG)
            mvec = jnp.maximum(mvec, cand)
        M = jnp.max(mvec)
        Ms = jnp.full((L,), M, jnp.float32)
        ivec = jnp.full((L,), BIG, jnp.int32)
        for j in range(nch):
            sl = pl.ds(j * L, L)
            cand = jnp.where((alive_v[sl] > 0) & (chosen_v[sl] == 0),
                             row_v[sl], NEG)
            hit = (cand == Ms) & (cand > -1.0e29)
            ivec = jnp.minimum(ivec, jnp.where(hit, iota + j * L, BIG))
        idx = jnp.min(ivec)
        idxs = jnp.full((L,), idx, jnp.int32)
        for j in range(nch):
            sl = pl.ds(j * L, L)
            pos = iota + j * L
            selv = pos == idxs
            nbv = (pos >= idxs - radius) & (pos <= idxs + radius)
            sel_f = selv.astype(jnp.float32)
            alive_v[sl] = jnp.where(nbv, sel_f, alive_v[sl])
            chosen_v[sl] = jnp.maximum(chosen_v[sl], sel_f)
        return carry

    lax.fori_loop(0, nstep, step, 0)

    for j in range(kslot // L):
        sl = pl.ds(j * L, L)
        cv_v[sl] = zeros
        ci_v[sl] = zeros
    rowc = r % C
    base = jnp.int32(0)
    for j in range(nch):
        sl = pl.ds(j * L, L)
        msk = chosen_v[sl] > 0
        chi = msk.astype(jnp.int32)
        slots = base + plsc.cumsum(chi) - 1
        pv = 1.0 / (1.0 + jnp.exp(-row_v[sl]))
        plsc.store_scatter(cv_v, [slots], pv, mask=msk)
        gidx = (rowc * T + iota + j * L).astype(jnp.float32)
        plsc.store_scatter(ci_v, [slots], gidx, mask=msk)
        base = base + jnp.sum(chi)
    pltpu.sync_copy(cv_v, cv_hbm.at[r])
    pltpu.sync_copy(ci_v, ci_hbm.at[r])


def _sc_nms_body(raw_hbm, cv_hbm, ci_hbm, row_v, alive_v, chosen_v, cv_v,
                 ci_v, *, R, T, C, radius, nstep, kslot):
    wid = lax.axis_index("c") * NS + lax.axis_index("s")
    nw = NC * NS
    do = functools.partial(_nms_one_row, raw_hbm=raw_hbm, cv_hbm=cv_hbm,
                           ci_hbm=ci_hbm, row_v=row_v, alive_v=alive_v,
                           chosen_v=chosen_v, cv_v=cv_v, ci_v=ci_v, T=T, C=C,
                           radius=radius, nstep=nstep, kslot=kslot)
    do(wid)
    if R > nw:
        @pl.when(wid < R - nw)
        def _():
            do(wid + nw)


def _sc_topp_body(cv_hbm, ci_hbm, masked_hbm, attn_hbm, cva, cia, cache_v,
                  masked_v, attn_v, *, B, CK, N, p):
    wid = lax.axis_index("c") * NS + lax.axis_index("s")
    ncc = CK // L
    nmc = N // L

    @pl.when(wid < B)
    def _():
        b = wid
        pltpu.sync_copy(cv_hbm.at[b], cva)
        pltpu.sync_copy(ci_hbm.at[b], cia)
        iota = lax.iota(jnp.int32, L)
        lane0 = iota == 0
        zeros = jnp.zeros((L,), jnp.float32)
        sv = zeros
        for j in range(ncc):
            sv = sv + cva[pl.ds(j * L, L)]
        denom_v = jnp.full((L,), jnp.sum(sv), jnp.float32) + 1e-8
        for h in range(-(-ncc // L)):
            cache_v[pl.ds(h * L, L)] = jnp.full((L,), NEG, jnp.float32)
        for j in range(ncc):
            mx = jnp.max(cva[pl.ds(j * L, L)])
            plsc.store_scatter(cache_v, [jnp.full((L,), j, jnp.int32)],
                               jnp.full((L,), mx, jnp.float32), mask=lane0)
        for j in range(nmc):
            masked_v[pl.ds(j * L, L)] = zeros

        def cond(c):
            return ~c[0]

        def body(c):
            _, count_v, cum_v, ssum_v = c
            c0 = cache_v[pl.ds(0, L)]
            c1 = cache_v[pl.ds(L, L)]
            M = jnp.max(jnp.maximum(c0, c1))
            Ms = jnp.full((L,), M, jnp.float32)
            j0 = jnp.min(jnp.where(c0 == Ms, iota, BIG))
            j1 = jnp.min(jnp.where(c1 == Ms, iota + L, BIG))
            jstar = jnp.minimum(j0, j1)
            cidx = jstar * L + iota
            ch = plsc.load_gather(cva, [cidx])
            lane = jnp.min(jnp.where(ch == Ms, iota, BIG))
            idxs = jnp.full((L,), jstar * L + lane, jnp.int32)
            gv = plsc.load_gather(cia, [idxs])
            gidx = jnp.min(gv).astype(jnp.int32)
            cum2 = cum_v + Ms / denom_v
            keep_v = ((cum2 <= p) | (count_v < 3)) & (Ms > -1.0e29)
            keep = jnp.min(keep_v.astype(jnp.int32)) > 0

            @pl.when(keep)
            def _():
                plsc.store_scatter(masked_v, [jnp.full((L,), gidx, jnp.int32)],
                                   Ms, mask=lane0)
                plsc.store_scatter(cva, [idxs],
                                   jnp.full((L,), NEG, jnp.float32),
                                   mask=lane0)
                ch2 = plsc.load_gather(cva, [cidx])
                plsc.store_scatter(cache_v,
                                   [jnp.full((L,), jstar, jnp.int32)],
                                   jnp.full((L,), jnp.max(ch2), jnp.float32),
                                   mask=lane0)

            return (~keep, count_v + 1, jnp.where(keep_v, cum2, cum_v),
                    jnp.where(keep_v, ssum_v + Ms, ssum_v))

        final = lax.while_loop(cond, body,
                               (jnp.bool_(False),
                                jnp.zeros((L,), jnp.int32),
                                jnp.zeros((L,), jnp.float32),
                                jnp.zeros((L,), jnp.float32)))
        d2 = final[3] + 1e-8
        for j in range(nmc):
            sl = pl.ds(j * L, L)
            attn_v[sl] = masked_v[sl] / d2
        pltpu.sync_copy(masked_v, masked_hbm.at[b])
        pltpu.sync_copy(attn_v, attn_hbm.at[b])


def _context_body(attn_ref, vals_ref, out_ref):
    out_ref[0] = lax.dot_general(
        attn_ref[0], vals_ref[0], (((1,), (0,)), ((), ())),
        preferred_element_type=jnp.float32,
        precision=lax.Precision.HIGHEST)


def kernel(query, values, W1_w, W1_b, V_w, V_b):
    B, C, T, D_in = query.shape
    D_hid = W1_w.shape[1]
    R = B * C
    N = C * T
    radius = int(round(0.05 * T))
    nstep = -(-T // (radius + 1))
    kslot = 32
    assert nstep <= kslot
    CK = C * kslot
    t_pad = -T % L
    Tp = T + t_pad
    f32 = jnp.float32

    q = query.reshape(R, T, D_in)
    rows_blk = 8

    raw = pl.pallas_call(
        functools.partial(_scores_body, rows=rows_blk, t_pad=t_pad),
        grid=(R // rows_blk,),
        in_specs=[
            pl.BlockSpec((rows_blk, T, D_in), lambda r: (r, 0, 0)),
            pl.BlockSpec((D_in, D_hid), lambda r: (0, 0)),
            pl.BlockSpec((1, D_hid), lambda r: (0, 0)),
            pl.BlockSpec((1, D_hid), lambda r: (0, 0)),
            pl.BlockSpec((1, 1), lambda r: (0, 0)),
        ],
        out_specs=pl.BlockSpec((rows_blk, Tp), lambda r: (r, 0)),
        out_shape=jax.ShapeDtypeStruct((R, Tp), f32),
    )(q, W1_w, W1_b.reshape(1, D_hid), V_w.reshape(1, D_hid),
      V_b.reshape(1, 1))

    mesh = plsc.VectorSubcoreMesh(core_axis_name="c", subcore_axis_name="s",
                                  num_cores=NC, num_subcores=NS)
    sc_params = pltpu.CompilerParams(needs_layout_passes=False)

    cv, ci = pl.kernel(
        functools.partial(_sc_nms_body, R=R, T=T, C=C, radius=radius,
                          nstep=nstep, kslot=kslot),
        out_type=(jax.ShapeDtypeStruct((R, kslot), f32),
                  jax.ShapeDtypeStruct((R, kslot), f32)),
        mesh=mesh,
        scratch_types=[
            pltpu.VMEM((Tp,), f32),
            pltpu.VMEM((Tp,), f32),
            pltpu.VMEM((Tp,), f32),
            pltpu.VMEM((kslot,), f32),
            pltpu.VMEM((kslot,), f32),
        ],
        compiler_params=sc_params,
    )(raw)

    masked, attn = pl.kernel(
        functools.partial(_sc_topp_body, B=B, CK=CK, N=N, p=0.7),
        out_type=(jax.ShapeDtypeStruct((B, N), f32),
                  jax.ShapeDtypeStruct((B, N), f32)),
        mesh=mesh,
        scratch_types=[
            pltpu.VMEM((CK,), f32),
            pltpu.VMEM((CK,), f32),
            pltpu.VMEM((2 * L,), f32),
            pltpu.VMEM((N,), f32),
            pltpu.VMEM((N,), f32),
        ],
        compiler_params=sc_params,
    )(cv.reshape(B, CK), ci.reshape(B, CK))

    context = pl.pallas_call(
        _context_body,
        grid=(B,),
        in_specs=[
            pl.BlockSpec((1, 1, N), lambda b: (b, 0, 0)),
            pl.BlockSpec((1, N, D_in), lambda b: (b, 0, 0)),
        ],
        out_specs=pl.BlockSpec((1, 1, D_in), lambda b: (b, 0, 0)),
        out_shape=jax.ShapeDtypeStruct((B, 1, D_in), f32),
    )(attn.reshape(B, 1, N), values.reshape(B, N, D_in))

    return (context.reshape(B, D_in), attn.reshape(B, C, T, 1),
            masked.reshape(B, C, T, 1))


# speedup vs baseline: 1.1043x; 1.1043x over previous
import functools

import jax
import jax.numpy as jnp
from jax import lax
from jax.experimental import pallas as pl
from jax.experimental.pallas import tpu as pltpu
from jax.experimental.pallas import tpu_sc as plsc

NC, NS, L = 2, 16, 16
NEG = -3.0e38
BIG = 10**9


def _scores_body(q_ref, w1_ref, b1_ref, vw_ref, vb_ref, out_ref, *, rows,
                 t_pad):
    for i in range(rows):
        h = jnp.tanh(jnp.dot(q_ref[i], w1_ref[...],
                             preferred_element_type=jnp.float32) + b1_ref[0])
        s = lax.dot_general(vw_ref[...], h, (((1,), (1,)), ((), ())),
                            preferred_element_type=jnp.float32)
        out_ref[i, :] = jnp.concatenate(
            [s[0] + vb_ref[0, 0], jnp.full((t_pad,), NEG, jnp.float32)])


def _nms_one_row(r, raw_hbm, cv_hbm, ci_hbm, row_v, raw_v, chosen_v, cv_v,
                 ci_v, *, T, C, radius, nstep, kslot):
    nch = row_v.shape[0] // L
    pltpu.sync_copy(raw_hbm.at[r], row_v)
    pltpu.sync_copy(raw_hbm.at[r], raw_v)
    iota = lax.iota(jnp.int32, L)
    ones = jnp.ones((L,), jnp.float32)
    zeros = jnp.zeros((L,), jnp.float32)
    lane0 = iota == 0
    cache = jnp.full((L,), NEG, jnp.float32)
    for j in range(nch):
        sl = pl.ds(j * L, L)
        chosen_v[sl] = zeros
        mx = jnp.max(row_v[sl])
        cache = jnp.where(iota == j, jnp.full((L,), mx, jnp.float32), cache)

    def cond(c):
        return c[0] > -1.0e29

    def body(c):
        M, cache = c
        Ms = jnp.full((L,), M, jnp.float32)
        jv = plsc.all_reduce_ffs(cache == Ms)
        ch = plsc.load_gather(row_v, [jv * L + iota])
        lv = plsc.all_reduce_ffs(ch == Ms)
        idxv = jv * L + lv
        plsc.store_scatter(chosen_v, [idxv], ones, mask=lane0)
        for dj in (-1, 0, 1):
            jn = jv + dj
            jc = jnp.clip(jn, 0, nch - 1)
            pos = jc * L + iota
            chn = ch if dj == 0 else plsc.load_gather(row_v, [pos])
            inwin = ((pos >= idxv - radius) & (pos <= idxv + radius)
                     & (jn == jc))
            new = jnp.where(inwin, NEG, chn)
            plsc.store_scatter(row_v, [pos], new)
            mx = jnp.max(new)
            cache = jnp.where(iota == jc, jnp.full((L,), mx, jnp.float32),
                              cache)
        return (jnp.max(cache), cache)

    lax.while_loop(cond, body, (jnp.max(cache), cache))

    for j in range(kslot // L):
        sl = pl.ds(j * L, L)
        cv_v[sl] = zeros
        ci_v[sl] = zeros
    rowc = r % C
    base = jnp.int32(0)
    for j in range(nch):
        sl = pl.ds(j * L, L)
        msk = chosen_v[sl] > 0
        chi = msk.astype(jnp.int32)
        slots = base + plsc.cumsum(chi) - 1
        pv = 1.0 / (1.0 + jnp.exp(-raw_v[sl]))
        plsc.store_scatter(cv_v, [slots], pv, mask=msk)
        gidx = (rowc * T + iota + j * L).astype(jnp.float32)
        plsc.store_scatter(ci_v, [slots], gidx, mask=msk)
        base = base + jnp.sum(chi)
    pltpu.sync_copy(cv_v, cv_hbm.at[r])
    pltpu.sync_copy(ci_v, ci_hbm.at[r])


def _sc_nms_body(raw_hbm, cv_hbm, ci_hbm, row_v, raw_v, chosen_v, cv_v,
                 ci_v, *, R, T, C, radius, nstep, kslot):
    wid = lax.axis_index("c") * NS + lax.axis_index("s")
    nw = NC * NS
    do = functools.partial(_nms_one_row, raw_hbm=raw_hbm, cv_hbm=cv_hbm,
                           ci_hbm=ci_hbm, row_v=row_v, raw_v=raw_v,
                           chosen_v=chosen_v, cv_v=cv_v, ci_v=ci_v, T=T, C=C,
                           radius=radius, nstep=nstep, kslot=kslot)
    do(wid)
    if R > nw:
        @pl.when(wid < R - nw)
        def _():
            do(wid + nw)


def _sc_topp_body(cv_hbm, ci_hbm, masked_hbm, attn_hbm, cva, cia,
                  masked_v, attn_v, *, B, CK, N, p):
    wid = lax.axis_index("c") * NS + lax.axis_index("s")
    ncc = CK // L
    nmc = N // L

    @pl.when(wid < B)
    def _():
        b = wid
        pltpu.sync_copy(cv_hbm.at[b], cva)
        pltpu.sync_copy(ci_hbm.at[b], cia)
        iota = lax.iota(jnp.int32, L)
        lane0 = iota == 0
        zeros = jnp.zeros((L,), jnp.float32)
        sv = zeros
        for j in range(ncc):
            sv = sv + cva[pl.ds(j * L, L)]
        denom_v = jnp.full((L,), jnp.sum(sv), jnp.float32) + 1e-8
        c0 = jnp.full((L,), NEG, jnp.float32)
        c1 = jnp.full((L,), NEG, jnp.float32)
        for j in range(ncc):
            mxv = jnp.full((L,), jnp.max(cva[pl.ds(j * L, L)]), jnp.float32)
            if j < L:
                c0 = jnp.where(iota == j, mxv, c0)
            else:
                c1 = jnp.where(iota == j - L, mxv, c1)
        for j in range(nmc):
            masked_v[pl.ds(j * L, L)] = zeros

        def cond(c):
            return ~c[0]

        def body(c):
            _, count_v, cum_v, ssum_v, c0, c1 = c
            M = jnp.max(jnp.maximum(c0, c1))
            Ms = jnp.full((L,), M, jnp.float32)
            h0 = c0 == Ms
            n0 = plsc.all_reduce_population_count(h0)
            j0 = plsc.all_reduce_ffs(h0)
            j1 = plsc.all_reduce_ffs(c1 == Ms)
            jv = jnp.where(n0 > 0, j0, j1 + L)
            cidx = jv * L + iota
            ch = plsc.load_gather(cva, [cidx])
            lv = plsc.all_reduce_ffs(ch == Ms)
            idxv = jv * L + lv
            gv = plsc.load_gather(cia, [idxv])
            cum2 = cum_v + Ms / denom_v
            keep_v = ((cum2 <= p) | (count_v < 3)) & (Ms > -1.0e29)
            keep = jnp.min(keep_v.astype(jnp.int32)) > 0
            newch = jnp.where(iota == lv, NEG, ch)
            mx2v = jnp.full((L,), jnp.max(newch), jnp.float32)
            c0n = jnp.where(keep_v & (jv < L) & (iota == jv), mx2v, c0)
            c1n = jnp.where(keep_v & (jv >= L) & (iota == jv - L), mx2v, c1)

            @pl.when(keep)
            def _():
                plsc.store_scatter(masked_v, [gv.astype(jnp.int32)], Ms,
                                   mask=lane0)
                plsc.store_scatter(cva, [cidx], newch)

            return (~keep, count_v + 1, jnp.where(keep_v, cum2, cum_v),
                    jnp.where(keep_v, ssum_v + Ms, ssum_v), c0n, c1n)

        final = lax.while_loop(cond, body,
                               (jnp.bool_(False),
                                jnp.zeros((L,), jnp.int32),
                                jnp.zeros((L,), jnp.float32),
                                jnp.zeros((L,), jnp.float32), c0, c1))
        d2 = final[3] + 1e-8
        for j in range(nmc):
            sl = pl.ds(j * L, L)
            attn_v[sl] = masked_v[sl] / d2
        pltpu.sync_copy(masked_v, masked_hbm.at[b])
        pltpu.sync_copy(attn_v, attn_hbm.at[b])


def _context_body(attn_ref, vals_ref, out_ref):
    out_ref[0] = lax.dot_general(
        attn_ref[0], vals_ref[0], (((1,), (0,)), ((), ())),
        preferred_element_type=jnp.float32,
        precision=lax.Precision.HIGHEST)


def kernel(query, values, W1_w, W1_b, V_w, V_b):
    B, C, T, D_in = query.shape
    D_hid = W1_w.shape[1]
    R = B * C
    N = C * T
    radius = int(round(0.05 * T))
    nstep = -(-T // (radius + 1))
    kslot = 32
    assert nstep <= kslot
    CK = C * kslot
    t_pad = -T % L
    Tp = T + t_pad
    f32 = jnp.float32

    q = query.reshape(R, T, D_in)
    rows_blk = 8

    raw = pl.pallas_call(
        functools.partial(_scores_body, rows=rows_blk, t_pad=t_pad),
        grid=(R // rows_blk,),
        in_specs=[
            pl.BlockSpec((rows_blk, T, D_in), lambda r: (r, 0, 0)),
            pl.BlockSpec((D_in, D_hid), lambda r: (0, 0)),
            pl.BlockSpec((1, D_hid), lambda r: (0, 0)),
            pl.BlockSpec((1, D_hid), lambda r: (0, 0)),
            pl.BlockSpec((1, 1), lambda r: (0, 0)),
        ],
        out_specs=pl.BlockSpec((rows_blk, Tp), lambda r: (r, 0)),
        out_shape=jax.ShapeDtypeStruct((R, Tp), f32),
    )(q, W1_w, W1_b.reshape(1, D_hid), V_w.reshape(1, D_hid),
      V_b.reshape(1, 1))

    mesh = plsc.VectorSubcoreMesh(core_axis_name="c", subcore_axis_name="s",
                                  num_cores=NC, num_subcores=NS)
    sc_params = pltpu.CompilerParams(needs_layout_passes=False)

    cv, ci = pl.kernel(
        functools.partial(_sc_nms_body, R=R, T=T, C=C, radius=radius,
                          nstep=nstep, kslot=kslot),
        out_type=(jax.ShapeDtypeStruct((R, kslot), f32),
                  jax.ShapeDtypeStruct((R, kslot), f32)),
        mesh=mesh,
        scratch_types=[
            pltpu.VMEM((Tp,), f32),
            pltpu.VMEM((Tp,), f32),
            pltpu.VMEM((Tp,), f32),
            pltpu.VMEM((kslot,), f32),
            pltpu.VMEM((kslot,), f32),
        ],
        compiler_params=sc_params,
    )(raw)

    masked, attn = pl.kernel(
        functools.partial(_sc_topp_body, B=B, CK=CK, N=N, p=0.7),
        out_type=(jax.ShapeDtypeStruct((B, N), f32),
                  jax.ShapeDtypeStruct((B, N), f32)),
        mesh=mesh,
        scratch_types=[
            pltpu.VMEM((CK,), f32),
            pltpu.VMEM((CK,), f32),
            pltpu.VMEM((N,), f32),
            pltpu.VMEM((N,), f32),
        ],
        compiler_params=sc_params,
    )(cv.reshape(B, CK), ci.reshape(B, CK))

    context = pl.pallas_call(
        _context_body,
        grid=(B,),
        in_specs=[
            pl.BlockSpec((1, 1, N), lambda b: (b, 0, 0)),
            pl.BlockSpec((1, N, D_in), lambda b: (b, 0, 0)),
        ],
        out_specs=pl.BlockSpec((1, 1, D_in), lambda b: (b, 0, 0)),
        out_shape=jax.ShapeDtypeStruct((B, 1, D_in), f32),
    )(attn.reshape(B, 1, N), values.reshape(B, N, D_in))

    return (context.reshape(B, D_in), attn.reshape(B, C, T, 1),
            masked.reshape(B, C, T, 1))
